# baseline (device time: 78906 ns/iter reference)
import contextlib
import os

import jax
import jax.numpy as jnp
from jax import lax
from jax.experimental import pallas as pl
from jax.experimental.pallas import tpu as pltpu

_PROF = os.environ.get("KPROF", "") == "1"
_NOCOMM = os.environ.get("KNOCOMM", "") == "1"


def _scope(name):
    return jax.named_scope(name) if _PROF else contextlib.nullcontext()


N_DEV = 4
WINDOW = 128
CDT = jnp.bfloat16


def kernel(x, Wq, K_ext, V_ext, Wo):
    B_loc, Sq, D = x.shape
    _, Hdim = Wq.shape
    Bg, Skv, Hq, Dh = K_ext.shape
    H_loc = Hdim // Dh

    me = lax.axis_index("i")

    k_arr = lax.dynamic_slice_in_dim(
        K_ext, B_loc * me, B_loc, axis=0).reshape(
            B_loc, Skv, Hq * Dh).astype(CDT)
    v_arr = lax.dynamic_slice_in_dim(
        V_ext, B_loc * me, B_loc, axis=0).reshape(
            B_loc, Skv, Hq * Dh).astype(CDT)

    def body(x_ref, wq_ref, k_ref, v_ref, wo_ref, out_ref,
             wq_comm, wo_comm, wq_send, wq_recv, wo_send, wo_recv):
        my = lax.axis_index("i")

        with _scope("barrier"):
            barrier = pltpu.get_barrier_semaphore()
            for d in range(1, N_DEV):
                pl.semaphore_signal(
                    barrier, inc=1,
                    device_id=(lax.rem(my + d, N_DEV),),
                    device_id_type=pl.DeviceIdType.MESH,
                )
            pl.semaphore_wait(barrier, N_DEV - 1)

        QC = Sq // 2
        KW = QC + WINDOW
        K0 = (0, Sq - KW)
        qi = lax.broadcasted_iota(jnp.int32, (QC, KW), 0)
        ki = lax.broadcasted_iota(jnp.int32, (QC, KW), 1)
        wins = [
            (jnp.abs(qi + c * QC - (ki + K0[c])) <= WINDOW
             ).astype(jnp.float32)
            for c in range(2)
        ]

        def compute_step(wq_w, wo_src, h):
            head_base = lax.rem(my - h + N_DEV, N_DEV) * (H_loc * Dh)
            ctxs = []
            for b in range(B_loc):
                q_full = (jnp.dot(x_ref[b], wq_w,
                                  preferred_element_type=jnp.float32)
                          * 0.125).astype(CDT)
                for c in range(2):
                    ctx_cols = []
                    for p in range(H_loc // 2):
                        off = head_base + p * (2 * Dh)
                        k2 = k_ref[b, K0[c]:K0[c] + KW, pl.ds(off, 2 * Dh)]
                        v2 = v_ref[b, K0[c]:K0[c] + KW, pl.ds(off, 2 * Dh)]
                        for half in range(2):
                            hl = 2 * p + half
                            q = q_full[c * QC:(c + 1) * QC,
                                       hl * Dh:(hl + 1) * Dh]
                            k = k2[:, half * Dh:(half + 1) * Dh]
                            v = v2[:, half * Dh:(half + 1) * Dh]
                            s = lax.dot_general(
                                q, k, (((1,), (1,)), ((), ())),
                                preferred_element_type=jnp.float32)
                            e = jnp.exp(s) * wins[c]
                            r = 1.0 / jnp.sum(e, axis=1, keepdims=True)
                            ctx_cols.append(
                                jnp.dot(e.astype(CDT), v,
                                        preferred_element_type=jnp.float32)
                                * r)
                    ctxs.append(
                        (b, c,
                         jnp.concatenate(ctx_cols, axis=1).astype(CDT)))
            wo_w = wo_src()
            for b, c, ctx in ctxs:
                contrib = jnp.dot(ctx, wo_w,
                                  preferred_element_type=jnp.float32)
                if h == 0:
                    out_ref[b, c * QC:(c + 1) * QC, :] = contrib
                else:
                    out_ref[b, c * QC:(c + 1) * QC, :] = (
                        out_ref[b, c * QC:(c + 1) * QC, :] + contrib)

        if _NOCOMM:
            for h in range(N_DEV):
                compute_step(wq_ref[...], lambda: wo_ref[...], h)
            return

        sends = []

        def start_send(d):
            dst = lax.rem(my + d, N_DEV)
            r_wq = pltpu.make_async_remote_copy(
                src_ref=wq_ref, dst_ref=wq_comm.at[d - 1],
                send_sem=wq_send.at[d - 1], recv_sem=wq_recv.at[d - 1],
                device_id=(dst,), device_id_type=pl.DeviceIdType.MESH)
            r_wo = pltpu.make_async_remote_copy(
                src_ref=wo_ref, dst_ref=wo_comm.at[d - 1],
                send_sem=wo_send.at[d - 1], recv_sem=wo_recv.at[d - 1],
                device_id=(dst,), device_id_type=pl.DeviceIdType.MESH)
            r_wq.start()
            r_wo.start()
            sends.extend([r_wq, r_wo])

        with _scope("send#d=1"):
            start_send(1)
        with _scope("compute#h=0"):
            compute_step(wq_ref[...], lambda: wo_ref[...], 0)
        for h in range(1, N_DEV):
            with _scope(f"wait_recv#h={h}"):
                pltpu.make_async_remote_copy(
                    src_ref=wq_ref, dst_ref=wq_comm.at[h - 1],
                    send_sem=wq_send.at[h - 1], recv_sem=wq_recv.at[h - 1],
                    device_id=(my,), device_id_type=pl.DeviceIdType.MESH,
                ).wait_recv()
            if h + 1 < N_DEV:
                with _scope(f"send#d={h + 1}"):
                    start_send(h + 1)

            def wo_src(h=h):
                pltpu.make_async_remote_copy(
                    src_ref=wo_ref, dst_ref=wo_comm.at[h - 1],
                    send_sem=wo_send.at[h - 1], recv_sem=wo_recv.at[h - 1],
                    device_id=(my,), device_id_type=pl.DeviceIdType.MESH,
                ).wait_recv()
                return wo_comm[h - 1]

            with _scope(f"compute#h={h}"):
                compute_step(wq_comm[h - 1], wo_src, h)
        with _scope("wait_sends"):
            for r in sends:
                r.wait_send()

    return pl.pallas_call(
        body,
        out_shape=jax.ShapeDtypeStruct((B_loc, Sq, D), jnp.float32),
        in_specs=[pl.BlockSpec(memory_space=pltpu.VMEM)] * 5,
        out_specs=pl.BlockSpec(memory_space=pltpu.VMEM),
        scratch_shapes=[
            pltpu.VMEM((N_DEV - 1, D, Hdim), CDT),
            pltpu.VMEM((N_DEV - 1, Hdim, D), CDT),
            pltpu.SemaphoreType.DMA((N_DEV - 1,)),
            pltpu.SemaphoreType.DMA((N_DEV - 1,)),
            pltpu.SemaphoreType.DMA((N_DEV - 1,)),
            pltpu.SemaphoreType.DMA((N_DEV - 1,)),
        ],
        compiler_params=pltpu.CompilerParams(
            collective_id=0, vmem_limit_bytes=100 * 1024 * 1024),
    )(x.astype(CDT), Wq.astype(CDT), k_arr, v_arr, Wo.astype(CDT))


# device time: 75895 ns/iter; 1.0397x vs baseline; 1.0397x over previous
import contextlib
import os

import jax
import jax.numpy as jnp
from jax import lax
from jax.experimental import pallas as pl
from jax.experimental.pallas import tpu as pltpu

_PROF = os.environ.get("KPROF", "") == "1"
_NOCOMM = os.environ.get("KNOCOMM", "") == "1"


def _scope(name):
    return jax.named_scope(name) if _PROF else contextlib.nullcontext()


N_DEV = 4
WINDOW = 128
CDT = jnp.bfloat16


def kernel(x, Wq, K_ext, V_ext, Wo):
    B_loc, Sq, D = x.shape
    _, Hdim = Wq.shape
    Bg, Skv, Hq, Dh = K_ext.shape
    H_loc = Hdim // Dh

    me = lax.axis_index("i")

    k_arr = lax.dynamic_slice_in_dim(
        K_ext, B_loc * me, B_loc, axis=0).reshape(
            B_loc, Skv, Hq * Dh).astype(CDT)
    v_arr = lax.dynamic_slice_in_dim(
        V_ext, B_loc * me, B_loc, axis=0).reshape(
            B_loc, Skv, Hq * Dh).astype(CDT)

    def body(x_ref, wq_ref, k_ref, v_ref, wo_ref, out_ref,
             wq_comm, wo_comm, wq_send, wq_recv, wo_send, wo_recv):
        my = lax.axis_index("i")

        with _scope("barrier"):
            barrier = pltpu.get_barrier_semaphore()
            for d in range(1, N_DEV):
                pl.semaphore_signal(
                    barrier, inc=1,
                    device_id=(lax.rem(my + d, N_DEV),),
                    device_id_type=pl.DeviceIdType.MESH,
                )
            pl.semaphore_wait(barrier, N_DEV - 1)

        QC = Sq // 2
        KW = QC + WINDOW
        K0 = (0, Sq - KW)
        qi = lax.broadcasted_iota(jnp.int32, (QC, KW), 0)
        ki = lax.broadcasted_iota(jnp.int32, (QC, KW), 1)
        wins = [
            (jnp.abs(qi + c * QC - (ki + K0[c])) <= WINDOW
             ).astype(jnp.float32)
            for c in range(2)
        ]

        def compute_step(wq_w, wo_src, h):
            head_base = lax.rem(my - h + N_DEV, N_DEV) * (H_loc * Dh)
            ctxs = []
            for b in range(B_loc):
                q_full = (jnp.dot(x_ref[b], wq_w,
                                  preferred_element_type=jnp.float32)
                          * 0.125).astype(CDT)
                for c in range(2):
                    ctx_cols = []
                    for p in range(H_loc // 2):
                        off = head_base + p * (2 * Dh)
                        k2 = k_ref[b, K0[c]:K0[c] + KW, pl.ds(off, 2 * Dh)]
                        v2 = v_ref[b, K0[c]:K0[c] + KW, pl.ds(off, 2 * Dh)]
                        for half in range(2):
                            hl = 2 * p + half
                            q = q_full[c * QC:(c + 1) * QC,
                                       hl * Dh:(hl + 1) * Dh]
                            k = k2[:, half * Dh:(half + 1) * Dh]
                            v = v2[:, half * Dh:(half + 1) * Dh]
                            s = lax.dot_general(
                                q, k, (((1,), (1,)), ((), ())),
                                preferred_element_type=jnp.float32)
                            e = jnp.exp(s) * wins[c]
                            r = 1.0 / jnp.sum(e, axis=1, keepdims=True)
                            ctx_cols.append(
                                jnp.dot(e.astype(CDT), v,
                                        preferred_element_type=jnp.float32)
                                * r)
                    ctxs.append(
                        (b, c,
                         jnp.concatenate(ctx_cols, axis=1).astype(CDT)))
            wo_w = wo_src()
            for b, c, ctx in ctxs:
                contrib = jnp.dot(ctx, wo_w,
                                  preferred_element_type=jnp.float32)
                if h == 0:
                    out_ref[b, c * QC:(c + 1) * QC, :] = contrib
                else:
                    out_ref[b, c * QC:(c + 1) * QC, :] = (
                        out_ref[b, c * QC:(c + 1) * QC, :] + contrib)

        if _NOCOMM:
            for h in range(N_DEV):
                compute_step(wq_ref[...], lambda: wo_ref[...], h)
            return

        sends = []

        def start_send(d):
            dst = lax.rem(my + d, N_DEV)
            r_wq = pltpu.make_async_remote_copy(
                src_ref=wq_ref, dst_ref=wq_comm.at[d - 1],
                send_sem=wq_send.at[d - 1], recv_sem=wq_recv.at[d - 1],
                device_id=(dst,), device_id_type=pl.DeviceIdType.MESH)
            r_wo = pltpu.make_async_remote_copy(
                src_ref=wo_ref, dst_ref=wo_comm.at[d - 1],
                send_sem=wo_send.at[d - 1], recv_sem=wo_recv.at[d - 1],
                device_id=(dst,), device_id_type=pl.DeviceIdType.MESH)
            r_wq.start()
            r_wo.start()
            sends.extend([r_wq, r_wo])

        with _scope("send_all"):
            for d in range(1, N_DEV):
                start_send(d)
        with _scope("compute#h=0"):
            compute_step(wq_ref[...], lambda: wo_ref[...], 0)
        for h in range(1, N_DEV):
            with _scope(f"wait_recv#h={h}"):
                pltpu.make_async_remote_copy(
                    src_ref=wq_ref, dst_ref=wq_comm.at[h - 1],
                    send_sem=wq_send.at[h - 1], recv_sem=wq_recv.at[h - 1],
                    device_id=(my,), device_id_type=pl.DeviceIdType.MESH,
                ).wait_recv()

            def wo_src(h=h):
                pltpu.make_async_remote_copy(
                    src_ref=wo_ref, dst_ref=wo_comm.at[h - 1],
                    send_sem=wo_send.at[h - 1], recv_sem=wo_recv.at[h - 1],
                    device_id=(my,), device_id_type=pl.DeviceIdType.MESH,
                ).wait_recv()
                return wo_comm[h - 1]

            with _scope(f"compute#h={h}"):
                compute_step(wq_comm[h - 1], wo_src, h)
        with _scope("wait_sends"):
            for r in sends:
                r.wait_send()

    return pl.pallas_call(
        body,
        out_shape=jax.ShapeDtypeStruct((B_loc, Sq, D), jnp.float32),
        in_specs=[pl.BlockSpec(memory_space=pltpu.VMEM)] * 5,
        out_specs=pl.BlockSpec(memory_space=pltpu.VMEM),
        scratch_shapes=[
            pltpu.VMEM((N_DEV - 1, D, Hdim), CDT),
            pltpu.VMEM((N_DEV - 1, Hdim, D), CDT),
            pltpu.SemaphoreType.DMA((N_DEV - 1,)),
            pltpu.SemaphoreType.DMA((N_DEV - 1,)),
            pltpu.SemaphoreType.DMA((N_DEV - 1,)),
            pltpu.SemaphoreType.DMA((N_DEV - 1,)),
        ],
        compiler_params=pltpu.CompilerParams(
            collective_id=0, vmem_limit_bytes=100 * 1024 * 1024),
    )(x.astype(CDT), Wq.astype(CDT), k_arr, v_arr, Wo.astype(CDT))
